# baseline (speedup 1.0000x reference)
import jax
import jax.numpy as jnp
from jax.experimental import pallas as pl
from jax.experimental.pallas import tpu as pltpu

N_NODES = 512
D_FEAT = 256
N_HIDDEN = 256
_GROUP = 8


def _dot(a, b):
    return jax.lax.dot_general(
        a, b, (((1,), (0,)), ((), ())),
        preferred_element_type=jnp.float32)


def _gcn_kernel(newf_ref, slot_ref, gidx_ref, *refs):
    G = _GROUP
    x_refs = refs[0:2 * G:2]
    a_refs = refs[1:2 * G:2]
    mask_ref = refs[2 * G]
    (Win_ref, bin_ref, W1_ref, b1_ref, W2_ref, b2_ref,
     W3_ref, b3_ref) = refs[2 * G + 1:2 * G + 9]
    out_ref = refs[2 * G + 9]
    h_scratch = refs[2 * G + 10]

    b = pl.program_id(0)
    news = [newf_ref[G * b + j] == 1 for j in range(G)]
    slots = [slot_ref[G * b + j] for j in range(G)]
    new_any = news[0]
    for j in range(1, G):
        new_any = jnp.logical_or(new_any, news[j])

    @pl.when(new_any)
    def _compute():
        bf = jnp.bfloat16
        x2 = jnp.concatenate([r[0] for r in x_refs], axis=0).astype(bf)
        a_bf = [r[0].astype(bf) for r in a_refs]
        h0 = jax.nn.relu(_dot(x2, Win_ref[...].astype(bf)) + bin_ref[...])
        h = h0
        for w_ref, b_ref, act in ((W1_ref, b1_ref, jax.nn.relu),
                                  (W2_ref, b2_ref, jax.nn.relu),
                                  (W3_ref, b3_ref, jnp.tanh)):
            hb = h.astype(bf)
            ts = [_dot(a_bf[j], hb[j * N_NODES:(j + 1) * N_NODES])
                  for j in range(G)]
            t = jnp.concatenate(ts, axis=0).astype(bf)
            h = act(_dot(t, w_ref[...].astype(bf)) + b_ref[...])
        hf = h + h0
        for j in range(G):
            h_scratch[slots[j]] = hf[j * N_NODES:(j + 1) * N_NODES]

    for j in range(G):
        m = mask_ref[j].astype(jnp.float32)
        out_ref[j] = _dot(m, h_scratch[slots[j]]) / jnp.maximum(
            jnp.sum(m), 1.0)


def kernel(graph, coverpoint_mask, batch_xs, batch_as, W_in, b_in,
           W1, b1, W2, b2, W3, b3):
    B = graph.shape[0]
    G = _GROUP
    g = graph.astype(jnp.int32)
    eq = g[:, None] == g[None, :]
    slot = jnp.argmax(eq, axis=1).astype(jnp.int32)
    newf = (slot == jnp.arange(B, dtype=jnp.int32)).astype(jnp.int32)
    mask_f = coverpoint_mask.reshape(B, 1, N_NODES)

    xa_specs = []
    for j in range(G):
        xa_specs.append(pl.BlockSpec(
            (1, N_NODES, D_FEAT),
            lambda b, nf, sl, gi, j=j: (gi[G * b + j], 0, 0)))
        xa_specs.append(pl.BlockSpec(
            (1, N_NODES, N_NODES),
            lambda b, nf, sl, gi, j=j: (gi[G * b + j], 0, 0)))
    mask_specs = [
        pl.BlockSpec((G, 1, N_NODES), lambda b, nf, sl, gi: (b, 0, 0))
    ]
    w_specs = []
    for shape in ((D_FEAT, N_HIDDEN), (1, N_HIDDEN)) * 4:
        w_specs.append(pl.BlockSpec(shape, lambda b, nf, sl, gi: (0, 0)))

    grid_spec = pltpu.PrefetchScalarGridSpec(
        num_scalar_prefetch=3,
        grid=(B // G,),
        in_specs=xa_specs + mask_specs + w_specs,
        out_specs=pl.BlockSpec((G, 1, N_HIDDEN),
                               lambda b, nf, sl, gi: (b, 0, 0)),
        scratch_shapes=[pltpu.VMEM((B, N_NODES, N_HIDDEN), jnp.float32)],
    )

    xa_args = []
    for j in range(G):
        xa_args += [batch_xs, batch_as]

    out = pl.pallas_call(
        _gcn_kernel,
        grid_spec=grid_spec,
        out_shape=jax.ShapeDtypeStruct((B, 1, N_HIDDEN), jnp.float32),
    )(newf, slot, g, *xa_args, mask_f,
      W_in, b_in.reshape(1, N_HIDDEN), W1, b1.reshape(1, N_HIDDEN),
      W2, b2.reshape(1, N_HIDDEN), W3, b3.reshape(1, N_HIDDEN))
    return out.reshape(B, N_HIDDEN)
